# baseline (speedup 1.0000x reference)
import functools

import jax
import jax.numpy as jnp
from jax import lax
from jax.experimental import pallas as pl
from jax.experimental.pallas import tpu as pltpu
from jax.experimental.pallas import tpu_sc as plsc

N = 50000
E = 800000
NBUS = 500
NITER = 11
NS = 16
LN = 16

NT = 3136
NP = NS * NT
ET = 52224
EP = NS * ET
CH = 1024
NCH = ET // CH
NBLK = 112


def _body(pk1, d1, pk2, d2, inp_h, den_h, bid_h, blk_h,
          theta_h, errp_h,
          theta_v, acc_s, u_s, pkb, dstb, valb, sem_in, sem_sc,
          inp_v, den_v, bid_v, acc_v, u_v, tn_v, ublk_v, blk_v, zer_v, err_v):
  t = lax.axis_index("s")
  nbase = t * NT

  pltpu.sync_copy(inp_h.at[pl.ds(nbase, NT)], inp_v)
  pltpu.sync_copy(den_h.at[pl.ds(nbase, NT)], den_v)
  pltpu.sync_copy(bid_h.at[pl.ds(nbase, NT)], bid_v)
  pltpu.sync_copy(blk_h, blk_v)

  def zer16(k, carry):
    for u in range(4):
      zer_v[pl.ds((k * 4 + u) * LN, LN)] = jnp.zeros((LN,), jnp.float32)
    return carry
  lax.fori_loop(0, NT // LN // 4, zer16, 0)

  def spmv(pk_h, dst_h):
    pltpu.sync_copy(zer_v, acc_s.at[pl.ds(nbase, NT)])
    plsc.subcore_barrier()
    pbase = t * (NCH * 2 * CH)
    ebase = t * ET

    def issue_in(c, s):
      pltpu.async_copy(
          pk_h.at[pl.ds(pbase + c * 2 * CH, 2 * CH)], pkb[s], sem_in[s])
      pltpu.async_copy(
          dst_h.at[pl.ds(ebase + c * CH, CH)], dstb[s], sem_in[s])

    def wait_in(c, s):
      pltpu.make_async_copy(
          pk_h.at[pl.ds(pbase + c * 2 * CH, 2 * CH)], pkb[s], sem_in[s]).wait()
      pltpu.make_async_copy(
          dst_h.at[pl.ds(ebase + c * CH, CH)], dstb[s], sem_in[s]).wait()

    def wait_sc(s):
      pltpu.make_async_copy(valb[s], acc_s.at[dstb[s]], sem_sc[s]).wait()

    issue_in(0, 0)

    def triple(p, carry):
      for b in range(3):
        c = p * 3 + b
        s = b
        sn = (b + 1) % 3
        wait_in(c, s)
        if b == 2:
          wait_sc(sn)
        else:
          @pl.when(p > 0)
          def _wsc():
            wait_sc(sn)
        if b == 2:
          @pl.when(p < NCH // 3 - 1)
          def _nxt():
            issue_in(c + 1, sn)
        else:
          issue_in(c + 1, sn)

        def g16(k16, carry2):
          for u in range(16):
            o = (k16 * 16 + u) * LN
            g = plsc.load_gather(theta_v, [pkb[s][pl.ds(o, LN)]])
            valb[s][pl.ds(o, LN)] = g * plsc.bitcast(
                pkb[s][pl.ds(CH + o, LN)], jnp.float32)
          return carry2
        lax.fori_loop(0, CH // LN // 16, g16, 0)
        pltpu.async_copy(valb[s], acc_s.at[dstb[s]], sem_sc[s], add=True)
      return carry
    lax.fori_loop(0, NCH // 3, triple, 0)
    wait_sc((NCH - 2) % 3)
    wait_sc((NCH - 1) % 3)
    plsc.subcore_barrier()

  def iter_body(i, carry):
    @pl.when(i > 0)
    def _run_gpg():
      spmv(pk1, d1)

    @pl.when(i == 0)
    def _zero_only():
      pltpu.sync_copy(zer_v, acc_s.at[pl.ds(nbase, NT)])
      plsc.subcore_barrier()

    pltpu.sync_copy(acc_s.at[pl.ds(nbase, NT)], acc_v)

    def e16(k, carry2):
      for u in range(4):
        sl = pl.ds((k * 4 + u) * LN, LN)
        u_v[sl] = (inp_v[sl] - acc_v[sl]) / den_v[sl]
      return carry2
    lax.fori_loop(0, NT // LN // 4, e16, 0)
    pltpu.sync_copy(u_v, u_s.at[pl.ds(nbase, NT)])
    plsc.subcore_barrier()

    pltpu.sync_copy(u_s.at[blk_v], ublk_v)

    def r16(k, carry2):
      for u in range(4):
        sl = pl.ds((k * 4 + u) * LN, LN)
        ub = plsc.load_gather(ublk_v, [bid_v[sl]])
        tn_v[sl] = u_v[sl] - ub
      return carry2
    lax.fori_loop(0, NT // LN // 4, r16, 0)
    pltpu.sync_copy(tn_v, theta_h.at[pl.ds(nbase, NT)])
    plsc.subcore_barrier()
    pltpu.sync_copy(theta_h, theta_v)

    spmv(pk2, d2)

    pltpu.sync_copy(acc_s.at[pl.ds(nbase, NT)], acc_v)

    def a16(k, vacc):
      for u in range(4):
        sl = pl.ds((k * 4 + u) * LN, LN)
        vacc = vacc + jnp.abs(inp_v[sl] - acc_v[sl])
      return vacc
    vacc = lax.fori_loop(0, NT // LN // 4, a16, jnp.zeros((LN,), jnp.float32))
    err_v[...] = jnp.full((LN,), jnp.sum(vacc), jnp.float32)
    pltpu.sync_copy(err_v, errp_h.at[i, t])
    return carry

  lax.fori_loop(0, NITER, iter_body, 0)


_mesh = plsc.VectorSubcoreMesh(
    core_axis_name="c", subcore_axis_name="s", num_cores=1)

_gpg = pl.kernel(
    _body,
    out_type=(
        jax.ShapeDtypeStruct((NP,), jnp.float32),
        jax.ShapeDtypeStruct((NITER, NS, LN), jnp.float32),
    ),
    mesh=_mesh,
    compiler_params=pltpu.CompilerParams(needs_layout_passes=False),
    scratch_types=[
        pltpu.VMEM((NP,), jnp.float32),
        pltpu.VMEM_SHARED((NP,), jnp.float32),
        pltpu.VMEM_SHARED((NP,), jnp.float32),
        [pltpu.VMEM((2 * CH,), jnp.int32)] * 3,
        [pltpu.VMEM((CH,), jnp.int32)] * 3,
        [pltpu.VMEM((CH,), jnp.float32)] * 3,
        [pltpu.SemaphoreType.DMA] * 3,
        [pltpu.SemaphoreType.DMA] * 3,
        pltpu.VMEM((NT,), jnp.float32),
        pltpu.VMEM((NT,), jnp.float32),
        pltpu.VMEM((NT,), jnp.int32),
        pltpu.VMEM((NT,), jnp.float32),
        pltpu.VMEM((NT,), jnp.float32),
        pltpu.VMEM((NT,), jnp.float32),
        pltpu.VMEM((NBLK,), jnp.float32),
        pltpu.VMEM((NBLK,), jnp.int32),
        pltpu.VMEM((NT,), jnp.float32),
        pltpu.VMEM((LN,), jnp.float32),
    ],
)


def _prep_edges(ei, ea):
  src = jnp.concatenate(
      [ei[0].astype(jnp.int32), jnp.zeros((EP - E,), jnp.int32)])
  dst = jnp.concatenate(
      [ei[1].astype(jnp.int32), jnp.zeros((EP - E,), jnp.int32)])
  w = jnp.concatenate(
      [ea.astype(jnp.float32) * 100.0, jnp.zeros((EP - E,), jnp.float32)])
  wbits = jax.lax.bitcast_convert_type(w, jnp.int32)
  pk = jnp.stack(
      [src.reshape(NS, NCH, CH), wbits.reshape(NS, NCH, CH)], axis=2)
  return pk.reshape(NS * NCH * 2 * CH), dst


@jax.jit
def kernel(x, y, edge_index_no_diag, edge_attr_no_diag, edge_index, edge_attr,
           ybus):
  del y
  inp = x[:, 0] - x[:, 1]
  den = jnp.diagonal(ybus, axis1=1, axis2=2).reshape(-1) * 100.0
  inp_p = jnp.concatenate([inp, jnp.zeros((NP - N,), jnp.float32)])
  den_p = jnp.concatenate([den, jnp.ones((NP - N,), jnp.float32)])
  ar = jnp.arange(NP, dtype=jnp.int32)
  bid = jnp.where(ar < N, ar // NBUS, 0).astype(jnp.int32)
  blk = (jnp.arange(NBLK, dtype=jnp.int32) % (N // NBUS)) * NBUS

  pk1, d1 = _prep_edges(edge_index_no_diag, edge_attr_no_diag)
  pk2, d2 = _prep_edges(edge_index, edge_attr)

  theta, errp = _gpg(pk1, d1, pk2, d2, inp_p, den_p, bid, blk)
  out = theta[:N].reshape(N, 1)
  errors = errp[:, :, 0].sum(axis=1)
  return (out, errors)


# speedup vs baseline: 1.1097x; 1.1097x over previous
import jax
import jax.numpy as jnp
from jax import lax
from jax.experimental import pallas as pl
from jax.experimental.pallas import tpu as pltpu
from jax.experimental.pallas import tpu_sc as plsc

N = 50000
E = 800000
NBUS = 500
NITER = 11
NC = 2
NS = 16
NW = NC * NS
LN = 16

NT = 1568
NP = NW * NT
NZ = NP // NS
ET = 26112
EP = NW * ET
CH = 512
NCH = ET // CH
NBLK = 112


def _body(pk1, d1, pk2, d2, inp_h, den_h, bid_h, blk_h,
          theta_h, errp_h, accp_h, u_h,
          theta_v, acc_s, pkb, dstb, valb, sem_in, sem_sc, gsem,
          inp_v, den_v, bid_v, acc_v, accb_v, u_v, tn_v, ublk_v, blk_v,
          zer_v, exp_v, err_v):
  cid = lax.axis_index("c")
  sid = lax.axis_index("s")
  wid = cid * NS + sid
  nbase = wid * NT
  zbase = sid * NZ

  def gbar():
    plsc.subcore_barrier()
    @pl.when(sid == 0)
    def _sg():
      pltpu.semaphore_signal(gsem, 1, core_index=1 - cid)
      pltpu.semaphore_wait(gsem, 1)
    plsc.subcore_barrier()

  pltpu.sync_copy(inp_h.at[pl.ds(nbase, NT)], inp_v)
  pltpu.sync_copy(den_h.at[pl.ds(nbase, NT)], den_v)
  pltpu.sync_copy(bid_h.at[pl.ds(nbase, NT)], bid_v)
  pltpu.sync_copy(blk_h, blk_v)

  def zer16(k, carry):
    for u in range(4):
      zer_v[pl.ds((k * 4 + u) * LN, LN)] = jnp.zeros((LN,), jnp.float32)
    return carry
  lax.fori_loop(0, NZ // LN // 4, zer16, 0)

  def zero_acc():
    pltpu.sync_copy(zer_v, acc_s.at[pl.ds(zbase, NZ)])
    plsc.subcore_barrier()

  def export_acc():
    plsc.subcore_barrier()
    pltpu.sync_copy(acc_s.at[pl.ds(zbase, NZ)], exp_v)
    pltpu.sync_copy(exp_v, accp_h.at[pl.ds(cid * NP + zbase, NZ)])
    gbar()

  def spmv(pk_h, dst_h):
    zero_acc()
    pbase = wid * (NCH * 2 * CH)
    ebase = wid * ET

    def issue_in(c, s):
      pltpu.async_copy(
          pk_h.at[pl.ds(pbase + c * 2 * CH, 2 * CH)], pkb[s], sem_in[s])
      pltpu.async_copy(
          dst_h.at[pl.ds(ebase + c * CH, CH)], dstb[s], sem_in[s])

    def wait_in(c, s):
      pltpu.make_async_copy(
          pk_h.at[pl.ds(pbase + c * 2 * CH, 2 * CH)], pkb[s], sem_in[s]).wait()
      pltpu.make_async_copy(
          dst_h.at[pl.ds(ebase + c * CH, CH)], dstb[s], sem_in[s]).wait()

    def wait_sc(s):
      pltpu.make_async_copy(valb[s], acc_s.at[dstb[s]], sem_sc[s]).wait()

    issue_in(0, 0)

    def triple(p, carry):
      for b in range(3):
        c = p * 3 + b
        s = b
        sn = (b + 1) % 3
        wait_in(c, s)
        if b == 2:
          wait_sc(sn)
        else:
          @pl.when(p > 0)
          def _wsc():
            wait_sc(sn)
        if b == 2:
          @pl.when(p < NCH // 3 - 1)
          def _nxt():
            issue_in(c + 1, sn)
        else:
          issue_in(c + 1, sn)

        def g16(k16, carry2):
          for u in range(16):
            o = (k16 * 16 + u) * LN
            g = plsc.load_gather(theta_v, [pkb[s][pl.ds(o, LN)]])
            valb[s][pl.ds(o, LN)] = g * plsc.bitcast(
                pkb[s][pl.ds(CH + o, LN)], jnp.float32)
          return carry2
        lax.fori_loop(0, CH // LN // 16, g16, 0)
        pltpu.async_copy(valb[s], acc_s.at[dstb[s]], sem_sc[s], add=True)
      return carry
    lax.fori_loop(0, NCH // 3, triple, 0)
    wait_sc((NCH - 2) % 3)
    wait_sc((NCH - 1) % 3)
    export_acc()

  def load_acc():
    pltpu.sync_copy(accp_h.at[pl.ds(nbase, NT)], acc_v)
    pltpu.sync_copy(accp_h.at[pl.ds(NP + nbase, NT)], accb_v)

  def iter_body(i, carry):
    @pl.when(i > 0)
    def _run_gpg():
      spmv(pk1, d1)

    @pl.when(i == 0)
    def _zero_only():
      zero_acc()
      export_acc()

    load_acc()

    def e16(k, carry2):
      for u in range(7):
        sl = pl.ds((k * 7 + u) * LN, LN)
        u_v[sl] = (inp_v[sl] - (acc_v[sl] + accb_v[sl])) / den_v[sl]
      return carry2
    lax.fori_loop(0, NT // LN // 7, e16, 0)
    pltpu.sync_copy(u_v, u_h.at[pl.ds(nbase, NT)])
    gbar()

    pltpu.sync_copy(u_h.at[blk_v], ublk_v)

    def r16(k, carry2):
      for u in range(7):
        sl = pl.ds((k * 7 + u) * LN, LN)
        ub = plsc.load_gather(ublk_v, [bid_v[sl]])
        tn_v[sl] = u_v[sl] - ub
      return carry2
    lax.fori_loop(0, NT // LN // 7, r16, 0)
    pltpu.sync_copy(tn_v, theta_h.at[pl.ds(nbase, NT)])
    gbar()
    pltpu.sync_copy(theta_h, theta_v)

    spmv(pk2, d2)

    load_acc()

    def a16(k, vacc):
      for u in range(7):
        sl = pl.ds((k * 7 + u) * LN, LN)
        vacc = vacc + jnp.abs(inp_v[sl] - (acc_v[sl] + accb_v[sl]))
      return vacc
    vacc = lax.fori_loop(0, NT // LN // 7, a16, jnp.zeros((LN,), jnp.float32))
    err_v[...] = jnp.full((LN,), jnp.sum(vacc), jnp.float32)
    pltpu.sync_copy(err_v, errp_h.at[i, wid])
    return carry

  lax.fori_loop(0, NITER, iter_body, 0)


_mesh = plsc.VectorSubcoreMesh(
    core_axis_name="c", subcore_axis_name="s", num_cores=2)

_gpg = pl.kernel(
    _body,
    out_type=(
        jax.ShapeDtypeStruct((NP,), jnp.float32),
        jax.ShapeDtypeStruct((NITER, NW, LN), jnp.float32),
        jax.ShapeDtypeStruct((NC * NP,), jnp.float32),
        jax.ShapeDtypeStruct((NP,), jnp.float32),
    ),
    mesh=_mesh,
    compiler_params=pltpu.CompilerParams(needs_layout_passes=False),
    scratch_types=[
        pltpu.VMEM((NP,), jnp.float32),
        pltpu.VMEM_SHARED((NP,), jnp.float32),
        [pltpu.VMEM((2 * CH,), jnp.int32)] * 3,
        [pltpu.VMEM((CH,), jnp.int32)] * 3,
        [pltpu.VMEM((CH,), jnp.float32)] * 3,
        [pltpu.SemaphoreType.DMA] * 3,
        [pltpu.SemaphoreType.DMA] * 3,
        pltpu.SemaphoreType.REGULAR,
        pltpu.VMEM((NT,), jnp.float32),
        pltpu.VMEM((NT,), jnp.float32),
        pltpu.VMEM((NT,), jnp.int32),
        pltpu.VMEM((NT,), jnp.float32),
        pltpu.VMEM((NT,), jnp.float32),
        pltpu.VMEM((NT,), jnp.float32),
        pltpu.VMEM((NT,), jnp.float32),
        pltpu.VMEM((NBLK,), jnp.float32),
        pltpu.VMEM((NBLK,), jnp.int32),
        pltpu.VMEM((NZ,), jnp.float32),
        pltpu.VMEM((NZ,), jnp.float32),
        pltpu.VMEM((LN,), jnp.float32),
    ],
)


def _prep_edges(ei, ea):
  src = jnp.concatenate(
      [ei[0].astype(jnp.int32), jnp.zeros((EP - E,), jnp.int32)])
  dst = jnp.concatenate(
      [ei[1].astype(jnp.int32), jnp.zeros((EP - E,), jnp.int32)])
  w = jnp.concatenate(
      [ea.astype(jnp.float32) * 100.0, jnp.zeros((EP - E,), jnp.float32)])
  wbits = jax.lax.bitcast_convert_type(w, jnp.int32)
  pk = jnp.stack(
      [src.reshape(NW, NCH, CH), wbits.reshape(NW, NCH, CH)], axis=2)
  return pk.reshape(NW * NCH * 2 * CH), dst


@jax.jit
def kernel(x, y, edge_index_no_diag, edge_attr_no_diag, edge_index, edge_attr,
           ybus):
  del y
  inp = x[:, 0] - x[:, 1]
  den = jnp.diagonal(ybus, axis1=1, axis2=2).reshape(-1) * 100.0
  inp_p = jnp.concatenate([inp, jnp.zeros((NP - N,), jnp.float32)])
  den_p = jnp.concatenate([den, jnp.ones((NP - N,), jnp.float32)])
  ar = jnp.arange(NP, dtype=jnp.int32)
  bid = jnp.where(ar < N, ar // NBUS, 0).astype(jnp.int32)
  blk = (jnp.arange(NBLK, dtype=jnp.int32) % (N // NBUS)) * NBUS

  pk1, d1 = _prep_edges(edge_index_no_diag, edge_attr_no_diag)
  pk2, d2 = _prep_edges(edge_index, edge_attr)

  theta, errp, _, _ = _gpg(pk1, d1, pk2, d2, inp_p, den_p, bid, blk)
  out = theta[:N].reshape(N, 1)
  errors = errp[:, :, 0].sum(axis=1)
  return (out, errors)
